# baseline (speedup 1.0000x reference)
import math
import functools

import jax
import jax.numpy as jnp
from jax import lax
from jax.experimental import pallas as pl
from jax.experimental.pallas import tpu as pltpu
from jax.experimental.pallas import tpu_sc as plsc

_NUM_POS_FEATS = 128
_TEMPERATURE = 10000.0
_LOG_T = math.log(_TEMPERATURE)

_SEQ = 8192
_FDIM = 1024

_NC, _NS, _L = 2, 16, 16
_NW = _NC * _NS
_RPW = _SEQ // _NW
_TAB_ROWS = _SEQ + _RPW + 8
_ZROWS = 16


def _table_body(o_ref):
    rows, feats = o_ref.shape
    pos = lax.broadcasted_iota(jnp.int32, (rows, 1), 0)
    j = lax.broadcasted_iota(jnp.int32, (1, feats), 1)
    j2 = (2 * (j // 2)).astype(jnp.float32) * (1.0 / _NUM_POS_FEATS)
    inv_dim = jnp.exp(-j2 * _LOG_T)
    phase = jnp.where(j % 2 == 0, 0.0, 0.5 * math.pi)
    theta = pos.astype(jnp.float32) * inv_dim + phase
    enc = jnp.sin(theta)
    o_ref[...] = jnp.where(pos < _SEQ, enc, 0.0)


def _make_table():
    return pl.pallas_call(
        _table_body,
        out_shape=jax.ShapeDtypeStruct((_TAB_ROWS, _NUM_POS_FEATS), jnp.float32),
    )()


def _sc_body(
    table_hbm, mask_hbm, out_hbm, mask_v, off_v, staged_v, rows_v, zbuf,
    sem_m, sem_g, sem_z,
):
    wid = lax.axis_index("s") * _NC + lax.axis_index("c")
    base = wid * _RPW

    mcpy = pltpu.async_copy(mask_hbm, mask_v, sem_m)
    zv = jnp.zeros((_L,), jnp.float32)
    for i in range(_ZROWS):
        for k in range((_FDIM - _NUM_POS_FEATS) // _L):
            zbuf[i, pl.ds(k * _L, _L)] = zv
    for k in range(_NUM_POS_FEATS // _L):
        staged_v[_RPW + 8, pl.ds(k * _L, _L)] = zv

    zcopies = [
        pltpu.async_copy(
            zbuf,
            out_hbm.at[
                pl.ds(base + r * _ZROWS, _ZROWS),
                pl.ds(_NUM_POS_FEATS, _FDIM - _NUM_POS_FEATS),
            ],
            sem_z,
        )
        for r in range(_RPW // _ZROWS)
    ]
    mcpy.wait()

    def _acc(i, a):
        for jj in range(_RPW // _L):
            a = a + mask_v[pl.ds(i * _RPW + jj * _L, _L)]
        return a

    acc = lax.fori_loop(0, wid, _acc, jnp.zeros((_L,), jnp.int32))
    prefix = jnp.sum(acc)

    pstart = pl.multiple_of((prefix // 8) * 8, 8)
    delta = prefix - pstart
    scpy = pltpu.async_copy(
        table_hbm.at[pl.ds(pstart, _RPW + 8)], staged_v.at[pl.ds(0, _RPW + 8)], sem_g
    )

    lcarry = jnp.int32(0)
    for c in range(_RPW // _L):
        v = mask_v[pl.ds(base + c * _L, _L)]
        cs = plsc.cumsum(v)
        off_v[pl.ds(c * _L, _L)] = jnp.where(v != 0, delta + lcarry + cs - 1, _RPW + 8)
        lcarry = lcarry + jnp.sum(v)
    scpy.wait()

    def _expand(c, _):
        ovec = off_v[pl.ds(c * _L, _L)]
        for l in range(_L):
            o = ovec[l]
            i = c * _L + l
            for k in range(_NUM_POS_FEATS // _L):
                rows_v[i, pl.ds(k * _L, _L)] = staged_v[o, pl.ds(k * _L, _L)]
        return 0

    lax.fori_loop(0, _RPW // _L, _expand, 0)

    rw = pltpu.async_copy(
        rows_v, out_hbm.at[pl.ds(base, _RPW), pl.ds(0, _NUM_POS_FEATS)], sem_z
    )
    for c in zcopies:
        c.wait()
    rw.wait()


_sc_scatter = functools.partial(
    pl.kernel,
    mesh=plsc.VectorSubcoreMesh(core_axis_name="c", subcore_axis_name="s"),
    out_type=jax.ShapeDtypeStruct((_SEQ, _FDIM), jnp.float32),
    scratch_types=[
        pltpu.VMEM((_SEQ,), jnp.int32),
        pltpu.VMEM((_RPW + _L,), jnp.int32),
        pltpu.VMEM((_RPW + 9, _NUM_POS_FEATS), jnp.float32),
        pltpu.VMEM((_RPW, _NUM_POS_FEATS), jnp.float32),
        pltpu.VMEM((_ZROWS, _FDIM - _NUM_POS_FEATS), jnp.float32),
        pltpu.SemaphoreType.DMA,
        pltpu.SemaphoreType.DMA,
        pltpu.SemaphoreType.DMA,
    ],
    compiler_params=pltpu.CompilerParams(needs_layout_passes=False),
)(_sc_body)


@jax.jit
def kernel(x, mask):
    bsz, seq_len, feature_dim = x.shape
    table = _make_table()
    out = _sc_scatter(table, mask.reshape(seq_len))
    return out.reshape(bsz, seq_len, feature_dim)


# speedup vs baseline: 1.1719x; 1.1719x over previous
import math
import functools

import jax
import jax.numpy as jnp
from jax import lax
from jax.experimental import pallas as pl
from jax.experimental.pallas import tpu as pltpu
from jax.experimental.pallas import tpu_sc as plsc

_NUM_POS_FEATS = 128
_TEMPERATURE = 10000.0
_LOG_T = math.log(_TEMPERATURE)

_SEQ = 8192
_FDIM = 1024

_NC, _NS, _L = 2, 16, 16
_NW = _NC * _NS
_RPW = _SEQ // _NW
_TBLK = 544
_TAB_ROWS = 16 * _TBLK
_RBLK = 512


def _tc_body(zero_ref, tab_ref):
    zero_ref[...] = jnp.zeros_like(zero_ref)
    g = pl.program_id(0)
    rows, feats = tab_ref.shape
    pos = g * _TBLK + lax.broadcasted_iota(jnp.int32, (rows, 1), 0)
    j = lax.broadcasted_iota(jnp.int32, (1, feats), 1)
    j2 = (2 * (j // 2)).astype(jnp.float32) * (1.0 / _NUM_POS_FEATS)
    inv_dim = jnp.exp(-j2 * _LOG_T)
    phase = jnp.where(j % 2 == 0, 0.0, 0.5 * math.pi)
    theta = pos.astype(jnp.float32) * inv_dim + phase
    tab_ref[...] = jnp.where(pos < _SEQ, jnp.sin(theta), 0.0)


def _make_zero_and_table():
    return pl.pallas_call(
        _tc_body,
        grid=(16,),
        out_specs=[
            pl.BlockSpec((_RBLK, _FDIM), lambda g: (g, 0)),
            pl.BlockSpec((_TBLK, _NUM_POS_FEATS), lambda g: (g, 0)),
        ],
        out_shape=[
            jax.ShapeDtypeStruct((_SEQ, _FDIM), jnp.float32),
            jax.ShapeDtypeStruct((_TAB_ROWS, _NUM_POS_FEATS), jnp.float32),
        ],
    )()


def _sc_body(
    table_hbm, mask_hbm, out_hbm, mask_v, off_v, staged_v, rows_v,
    sem_m, sem_g, sem_z,
):
    wid = lax.axis_index("s") * _NC + lax.axis_index("c")
    base = wid * _RPW

    mcpy = pltpu.async_copy(mask_hbm, mask_v, sem_m)
    zv = jnp.zeros((_L,), jnp.float32)
    for k in range(_NUM_POS_FEATS // _L):
        staged_v[_RPW + 8, pl.ds(k * _L, _L)] = zv
    mcpy.wait()

    def _acc(i, a):
        for jj in range(_RPW // _L):
            a = a + mask_v[pl.ds(i * _RPW + jj * _L, _L)]
        return a

    acc = lax.fori_loop(0, wid, _acc, jnp.zeros((_L,), jnp.int32))
    prefix = jnp.sum(acc)

    pstart = pl.multiple_of((prefix // 8) * 8, 8)
    delta = prefix - pstart
    scpy = pltpu.async_copy(
        table_hbm.at[pl.ds(pstart, _RPW + 8)], staged_v.at[pl.ds(0, _RPW + 8)], sem_g
    )

    lcarry = jnp.int32(0)
    for c in range(_RPW // _L):
        v = mask_v[pl.ds(base + c * _L, _L)]
        cs = plsc.cumsum(v)
        off_v[pl.ds(c * _L, _L)] = jnp.where(v != 0, delta + lcarry + cs - 1, _RPW + 8)
        lcarry = lcarry + jnp.sum(v)
    scpy.wait()

    def _expand(c, _):
        ovec = off_v[pl.ds(c * _L, _L)]
        for l in range(_L):
            o = ovec[l]
            i = c * _L + l
            for k in range(_NUM_POS_FEATS // _L):
                rows_v[i, pl.ds(k * _L, _L)] = staged_v[o, pl.ds(k * _L, _L)]
        return 0

    lax.fori_loop(0, _RPW // _L, _expand, 0)

    pltpu.async_copy(
        rows_v, out_hbm.at[pl.ds(base, _RPW), pl.ds(0, _NUM_POS_FEATS)], sem_z
    ).wait()


_sc_fill = functools.partial(
    pl.kernel,
    mesh=plsc.VectorSubcoreMesh(core_axis_name="c", subcore_axis_name="s"),
    out_type=(),
    scratch_types=[
        pltpu.VMEM((_SEQ,), jnp.int32),
        pltpu.VMEM((_RPW + _L,), jnp.int32),
        pltpu.VMEM((_RPW + 9, _NUM_POS_FEATS), jnp.float32),
        pltpu.VMEM((_RPW, _NUM_POS_FEATS), jnp.float32),
        pltpu.SemaphoreType.DMA,
        pltpu.SemaphoreType.DMA,
        pltpu.SemaphoreType.DMA,
    ],
    compiler_params=pltpu.CompilerParams(needs_layout_passes=False),
)(_sc_body)


@jax.jit
def kernel(x, mask):
    bsz, seq_len, feature_dim = x.shape
    out0, table = _make_zero_and_table()
    out_ref = jax.new_ref(out0)
    _sc_fill(table, mask.reshape(seq_len), out_ref)
    out = jax.freeze(out_ref)
    return out.reshape(bsz, seq_len, feature_dim)


# speedup vs baseline: 1.1721x; 1.0002x over previous
import math
import functools

import jax
import jax.numpy as jnp
from jax import lax
from jax.experimental import pallas as pl
from jax.experimental.pallas import tpu as pltpu
from jax.experimental.pallas import tpu_sc as plsc

_NUM_POS_FEATS = 128
_TEMPERATURE = 10000.0
_LOG_T = math.log(_TEMPERATURE)

_SEQ = 8192
_FDIM = 1024

_NC, _NS, _L = 2, 16, 16
_NW = _NC * _NS
_RPW = _SEQ // _NW
_TBLK = 544
_TAB_ROWS = 16 * _TBLK
_RBLK = 512


def _tc_body(zero_ref, tab_ref):
    zero_ref[...] = jnp.zeros_like(zero_ref)
    g = pl.program_id(0)
    rows, feats = tab_ref.shape
    pos = g * _TBLK + lax.broadcasted_iota(jnp.int32, (rows, 1), 0)
    j = lax.broadcasted_iota(jnp.int32, (1, feats), 1)
    j2 = (2 * (j // 2)).astype(jnp.float32) * (1.0 / _NUM_POS_FEATS)
    inv_dim = jnp.exp(-j2 * _LOG_T)
    phase = jnp.where(j % 2 == 0, 0.0, 0.5 * math.pi)
    theta = pos.astype(jnp.float32) * inv_dim + phase
    tab_ref[...] = jnp.where(pos < _SEQ, jnp.sin(theta), 0.0)


def _make_zero_and_table():
    return pl.pallas_call(
        _tc_body,
        grid=(16,),
        out_specs=[
            pl.BlockSpec((_RBLK, _FDIM), lambda g: (g, 0)),
            pl.BlockSpec((_TBLK, _NUM_POS_FEATS), lambda g: (g, 0)),
        ],
        out_shape=[
            jax.ShapeDtypeStruct((_SEQ, _FDIM), jnp.float32),
            jax.ShapeDtypeStruct((_TAB_ROWS, _NUM_POS_FEATS), jnp.float32),
        ],
    )()


def _sc_body(
    table_hbm, mask_hbm, out_hbm, mask_v, off_v, staged_v, rows_v,
    sem_m, sem_g, sem_z,
):
    wid = lax.axis_index("s") * _NC + lax.axis_index("c")
    base = wid * _RPW

    mcpy = pltpu.async_copy(mask_hbm, mask_v, sem_m)
    zv = jnp.zeros((_L,), jnp.float32)
    for k in range(_NUM_POS_FEATS // _L):
        staged_v[_RPW + 8, pl.ds(k * _L, _L)] = zv
    mcpy.wait()

    def _acc(i, a):
        for jj in range(_RPW // _L):
            a = a + mask_v[pl.ds(i * _RPW + jj * _L, _L)]
        return a

    acc = lax.fori_loop(0, wid, _acc, jnp.zeros((_L,), jnp.int32))
    prefix = jnp.sum(acc)

    pstart = pl.multiple_of((prefix // 8) * 8, 8)
    delta = prefix - pstart
    scpy = pltpu.async_copy(
        table_hbm.at[pl.ds(pstart, _RPW + 8)], staged_v.at[pl.ds(0, _RPW + 8)], sem_g
    )

    lcarry = jnp.int32(0)
    for c in range(_RPW // _L):
        v = mask_v[pl.ds(base + c * _L, _L)]
        cs = plsc.cumsum(v)
        off_v[pl.ds(c * _L, _L)] = jnp.where(v != 0, delta + lcarry + cs - 1, _RPW + 8)
        lcarry = lcarry + jnp.sum(v)
    scpy.wait()

    def _expand(c, _):
        ovec = off_v[pl.ds(c * _L, _L)]
        for l in range(_L):
            o = ovec[l]
            i = c * _L + l
            for k in range(_NUM_POS_FEATS // _L):
                rows_v[i, pl.ds(k * _L, _L)] = staged_v[o, pl.ds(k * _L, _L)]
        return 0

    half = _RPW // 2
    lax.fori_loop(0, half // _L, _expand, 0)
    w1 = pltpu.async_copy(
        rows_v.at[pl.ds(0, half)],
        out_hbm.at[pl.ds(base, half), pl.ds(0, _NUM_POS_FEATS)],
        sem_z,
    )
    lax.fori_loop(half // _L, _RPW // _L, _expand, 0)
    w2 = pltpu.async_copy(
        rows_v.at[pl.ds(half, half)],
        out_hbm.at[pl.ds(base + half, half), pl.ds(0, _NUM_POS_FEATS)],
        sem_z,
    )
    w1.wait()
    w2.wait()


_sc_fill = functools.partial(
    pl.kernel,
    mesh=plsc.VectorSubcoreMesh(core_axis_name="c", subcore_axis_name="s"),
    out_type=(),
    scratch_types=[
        pltpu.VMEM((_SEQ,), jnp.int32),
        pltpu.VMEM((_RPW + _L,), jnp.int32),
        pltpu.VMEM((_RPW + 9, _NUM_POS_FEATS), jnp.float32),
        pltpu.VMEM((_RPW, _NUM_POS_FEATS), jnp.float32),
        pltpu.SemaphoreType.DMA,
        pltpu.SemaphoreType.DMA,
        pltpu.SemaphoreType.DMA,
    ],
    compiler_params=pltpu.CompilerParams(needs_layout_passes=False),
)(_sc_body)


@jax.jit
def kernel(x, mask):
    bsz, seq_len, feature_dim = x.shape
    out0, table = _make_zero_and_table()
    out_ref = jax.new_ref(out0)
    _sc_fill(table, mask.reshape(seq_len), out_ref)
    out = jax.freeze(out_ref)
    return out.reshape(bsz, seq_len, feature_dim)
